# baseline (speedup 1.0000x reference)
import functools

import jax
import jax.numpy as jnp
from jax.experimental import pallas as pl
from jax.experimental.pallas import tpu as pltpu
from jax.experimental.pallas import tpu_sc as plsc

B = 8
N = 16384
G = 512
K = 32
BG = 256
BG3 = 512
NCH = 8
NC = N // NCH
GS = 32
NG = N // GS
GPC = NC // GS
NSEL = 36
CW = NSEL * GS
BIG_I = 1 << 30

SC_CORES = 2
SC_SUBCORES = 16
NW = SC_CORES * SC_SUBCORES
ROWS_W = (B * G) // NW
PW = (B * G * K) // NW
GW = G // (NW // B)


def _k1_body(xt_ref, c_ref, d2_ref, gsel_ref, srow_ref, q_ref):
    b = pl.program_id(0)
    gb = pl.program_id(1)
    c = c_ref[0]
    c0 = c[:, 0:1]
    c1 = c[:, 1:2]
    c2 = c[:, 2:3]
    csq = c0 * c0 + c1 * c1 + c2 * c2
    cb16 = c.astype(jnp.bfloat16)
    inf = jnp.float32(jnp.inf)

    rowi = jax.lax.broadcasted_iota(jnp.int32, (NC, GPC), 0)
    coli = jax.lax.broadcasted_iota(jnp.int32, (NC, GPC), 1)
    sel = (rowi == coli * GS).astype(jnp.float32)

    mins = []
    for j in range(NCH):
        xc = xt_ref[0, j]
        x0 = xc[0:1, :]
        x1 = xc[1:2, :]
        x2 = xc[2:3, :]
        xsq = x0 * x0 + x1 * x1 + x2 * x2
        dot = jax.lax.dot_general(
            cb16, xc.astype(jnp.bfloat16), (((1,), (0,)), ((), ())),
            preferred_element_type=jnp.float32)
        d2 = (csq - 2.0 * dot) + xsq
        for h in range(NC // 128):
            d2_ref[0, j * (NC // 128) + h, :, :] = d2[:, h * 128:(h + 1) * 128]
        m = d2
        for sh in (1, 2, 4, 8, 16):
            m = jnp.minimum(m, pltpu.roll(m, NC - sh, 1))
        mins.append(jax.lax.dot_general(
            m, sel, (((1,), (0,)), ((), ())),
            preferred_element_type=jnp.float32,
            precision=jax.lax.Precision.HIGHEST))
    gmin = jnp.concatenate(mins, axis=1)

    cid = jax.lax.broadcasted_iota(jnp.int32, (BG, NG), 1)
    kiota = jax.lax.broadcasted_iota(jnp.int32, (BG, NSEL), 1)

    def select_grp(k, carry):
        mprev, iprev, acc = carry
        valid = (gmin > mprev) | ((gmin == mprev) & (cid > iprev))
        dm = jnp.where(valid, gmin, inf)
        mc = jnp.min(dm, axis=1, keepdims=True)
        ic = jnp.min(jnp.where(dm == mc, cid, BIG_I), axis=1, keepdims=True)
        acc = jnp.where(kiota == k, jnp.broadcast_to(ic, (BG, NSEL)), acc)
        return (mc, ic, acc)

    mprev0 = jnp.full((BG, 1), -jnp.inf, jnp.float32)
    iprev0 = jnp.full((BG, 1), -1, jnp.int32)
    acc0 = jnp.zeros((BG, NSEL), jnp.int32)
    _, _, acc = jax.lax.fori_loop(0, NSEL, select_grp, (mprev0, iprev0, acc0))
    gsel_ref[0] = acc
    g_row = gb * BG + jax.lax.broadcasted_iota(jnp.int32, (BG, NSEL), 0)
    srow_ref[0] = (b * (N // 128) + (acc >> 2)) * G + g_row
    q_ref[0] = acc & 3


def _k1_call(xt_c, center):
    return pl.pallas_call(
        _k1_body,
        grid=(B, G // BG),
        in_specs=[
            pl.BlockSpec((1, NCH, 3, NC), lambda b, g: (b, 0, 0, 0)),
            pl.BlockSpec((1, BG, 3), lambda b, g: (b, g, 0)),
        ],
        out_specs=[
            pl.BlockSpec((1, N // 128, BG, 128), lambda b, g: (b, 0, g, 0)),
            pl.BlockSpec((1, BG, NSEL), lambda b, g: (b, g, 0)),
            pl.BlockSpec((1, BG, NSEL), lambda b, g: (b, g, 0)),
            pl.BlockSpec((1, BG, NSEL), lambda b, g: (b, g, 0)),
        ],
        out_shape=[
            jax.ShapeDtypeStruct((B, N // 128, G, 128), jnp.float32),
            jax.ShapeDtypeStruct((B, G, NSEL), jnp.int32),
            jax.ShapeDtypeStruct((B, G, NSEL), jnp.int32),
            jax.ShapeDtypeStruct((B, G, NSEL), jnp.int32),
        ],
        compiler_params=pltpu.CompilerParams(
            dimension_semantics=("parallel", "parallel"),
        ),
    )(xt_c, center)


def _k2_body(tab_hbm, idx_hbm, q_hbm, out_hbm, idx_v, q_v,
             dest_0, dest_1, dest_2, dest_3, stg_a, stg_b,
             sem_0, sem_1, sem_2, sem_3):
    cid = jax.lax.axis_index("c")
    sid = jax.lax.axis_index("s")
    wid = sid * SC_CORES + cid
    rounds = (ROWS_W * NSEL) // 128
    nw = ROWS_W * NSEL
    dests = (dest_0, dest_1, dest_2, dest_3)
    sems = (sem_0, sem_1, sem_2, sem_3)

    pltpu.sync_copy(idx_hbm.at[pl.ds(wid * nw, nw)], idx_v)
    pltpu.sync_copy(q_hbm.at[pl.ds(wid * nw, nw)], q_v)

    lanes = jax.lax.iota(jnp.int32, 16)

    def start(r, i):
        off = jnp.minimum(r, rounds - 1) * 128
        return pltpu.async_copy(
            tab_hbm.at[idx_v.at[pl.ds(off, 128)]], dests[i], sems[i])

    def trim(r, dest, stg):
        for sg in range(8):
            slots = sg * 16 + lanes
            q16 = q_v[pl.ds(r * 128 + sg * 16, 16)]
            col0 = q16 * GS
            base = slots * GS
            for s in range(GS):
                vals = plsc.load_gather(dest, [slots, col0 + s])
                plsc.store_scatter(stg, [base + s], vals)

    for i in range(4):
        start(jnp.int32(i), i)

    def quad(r4, carry):
        r0 = r4 * 4
        for i in range(4):
            r = r0 + i
            pltpu.make_async_copy(
                tab_hbm.at[idx_v.at[pl.ds(jnp.minimum(r, rounds - 1) * 128,
                                          128)]],
                dests[i], sems[i]).wait()
            stg = stg_a if i % 2 == 0 else stg_b
            trim(r, dests[i], stg)
            pltpu.sync_copy(
                stg,
                out_hbm.at[pl.ds((wid * rounds + r) * 128 * GS, 128 * GS)])
            start(r + 4, i)
        return carry

    jax.lax.fori_loop(0, rounds // 4, quad, 0)
    for i in range(4):
        pltpu.make_async_copy(
            tab_hbm.at[idx_v.at[pl.ds((rounds - 1) * 128, 128)]],
            dests[i], sems[i]).wait()


@functools.cache
def _k2_call():
    return pl.kernel(
        _k2_body,
        out_type=jax.ShapeDtypeStruct((B * G * CW,), jnp.float32),
        mesh=plsc.VectorSubcoreMesh(
            core_axis_name="c", subcore_axis_name="s",
            num_cores=SC_CORES, num_subcores=SC_SUBCORES),
        compiler_params=pltpu.CompilerParams(needs_layout_passes=False),
        scratch_types=[
            pltpu.VMEM((ROWS_W * NSEL,), jnp.int32),
            pltpu.VMEM((ROWS_W * NSEL,), jnp.int32),
            pltpu.VMEM((128, 128), jnp.float32),
            pltpu.VMEM((128, 128), jnp.float32),
            pltpu.VMEM((128, 128), jnp.float32),
            pltpu.VMEM((128, 128), jnp.float32),
            pltpu.VMEM((128 * GS,), jnp.float32),
            pltpu.VMEM((128 * GS,), jnp.float32),
            pltpu.SemaphoreType.DMA,
            pltpu.SemaphoreType.DMA,
            pltpu.SemaphoreType.DMA,
            pltpu.SemaphoreType.DMA,
        ],
    )


def _k3_body(cand_ref, gsel_ref, idx_ref, orig_ref):
    inf = jnp.float32(jnp.inf)
    gsel = gsel_ref[0]
    siota = jax.lax.broadcasted_iota(jnp.int32, (BG3, GS), 1)
    for t in range(NSEL):
        orig_ref[:, t * GS:(t + 1) * GS] = gsel[:, t:t + 1] * GS + siota

    cand = cand_ref[0]
    orig = orig_ref[...]
    kiota = jax.lax.broadcasted_iota(jnp.int32, (BG3, K), 1)

    def select_k(k, carry):
        mprev, iprev, acc = carry
        valid = (cand > mprev) | ((cand == mprev) & (orig > iprev))
        dm = jnp.where(valid, cand, inf)
        mc = jnp.min(dm, axis=1, keepdims=True)
        ic = jnp.min(jnp.where(dm == mc, orig, BIG_I), axis=1, keepdims=True)
        acc = jnp.where(kiota == k, jnp.broadcast_to(ic, (BG3, K)), acc)
        return (mc, ic, acc)

    mprev0 = jnp.full((BG3, 1), -jnp.inf, jnp.float32)
    iprev0 = jnp.full((BG3, 1), -1, jnp.int32)
    acc0 = jnp.zeros((BG3, K), jnp.int32)
    _, _, acc = jax.lax.fori_loop(0, K, select_k, (mprev0, iprev0, acc0))
    idx_ref[0] = acc


def _k3_call(cand, gsel):
    return pl.pallas_call(
        _k3_body,
        grid=(B, G // BG3),
        in_specs=[
            pl.BlockSpec((1, BG3, CW), lambda b, g: (b, g, 0)),
            pl.BlockSpec((1, BG3, NSEL), lambda b, g: (b, g, 0)),
        ],
        out_specs=pl.BlockSpec((1, BG3, K), lambda b, g: (b, g, 0)),
        out_shape=jax.ShapeDtypeStruct((B, G, K), jnp.int32),
        scratch_shapes=[pltpu.VMEM((BG3, CW), jnp.int32)],
        compiler_params=pltpu.CompilerParams(
            dimension_semantics=("parallel", "parallel"),
        ),
    )(cand, gsel)


def _k4_body(xyz_hbm, idx_hbm, cen_hbm, out_hbm, xyz_v, idx_v, cen_v, out_v):
    cid = jax.lax.axis_index("c")
    sid = jax.lax.axis_index("s")
    wid = sid * SC_CORES + cid
    b = wid // (NW // B)
    gc = wid % (NW // B)

    pltpu.sync_copy(xyz_hbm.at[pl.ds(b * (N * 3), N * 3)], xyz_v)
    pltpu.sync_copy(idx_hbm.at[pl.ds(wid * PW, PW)], idx_v)
    pltpu.sync_copy(
        cen_hbm.at[pl.ds(b * (G * 3) + gc * (GW * 3), GW * 3)], cen_v)

    lanes = jax.lax.iota(jnp.int32, 16)

    def step(i, carry):
        base = i * 16
        iv = idx_v[pl.ds(base, 16)]
        lf = base + lanes
        g3 = jax.lax.shift_right_logical(lf, 5) * 3
        a3 = iv * 3
        o3 = lf * 3
        for d in range(3):
            p = plsc.load_gather(xyz_v, [a3 + d])
            c = plsc.load_gather(cen_v, [g3 + d])
            plsc.store_scatter(out_v, [o3 + d], p - c)
        return carry

    jax.lax.fori_loop(0, PW // 16, step, 0)

    pltpu.sync_copy(out_v, out_hbm.at[pl.ds(wid * (PW * 3), PW * 3)])


@functools.cache
def _k4_call():
    return pl.kernel(
        _k4_body,
        out_type=jax.ShapeDtypeStruct((B * G * K * 3,), jnp.float32),
        mesh=plsc.VectorSubcoreMesh(
            core_axis_name="c", subcore_axis_name="s",
            num_cores=SC_CORES, num_subcores=SC_SUBCORES),
        compiler_params=pltpu.CompilerParams(needs_layout_passes=False),
        scratch_types=[
            pltpu.VMEM((N * 3,), jnp.float32),
            pltpu.VMEM((PW,), jnp.int32),
            pltpu.VMEM((GW * 3,), jnp.float32),
            pltpu.VMEM((PW * 3,), jnp.float32),
        ],
    )


@jax.jit
def kernel(xyz, center):
    xt = jnp.swapaxes(xyz, 1, 2)
    xt_c = jnp.swapaxes(xt.reshape(B, 3, NCH, NC), 1, 2)
    d2lin, gsel, srow, q = _k1_call(xt_c, center)
    cand = _k2_call()(d2lin.reshape(B * (N // 128) * G, 128),
                      srow.reshape(B * G * NSEL),
                      q.reshape(B * G * NSEL))
    cand = cand.reshape(B, G, CW)
    idx = _k3_call(cand, gsel)
    out = _k4_call()(xyz.reshape(B * N * 3),
                     idx.reshape(B * G * K),
                     center.reshape(B * G * 3))
    return out.reshape(B, G, K, 3)


# speedup vs baseline: 1.0086x; 1.0086x over previous
import functools

import jax
import jax.numpy as jnp
from jax.experimental import pallas as pl
from jax.experimental.pallas import tpu as pltpu
from jax.experimental.pallas import tpu_sc as plsc

B = 8
N = 16384
G = 512
K = 32
BG = 256
BG3 = 512
NCH = 8
NC = N // NCH
GS = 32
NG = N // GS
GPC = NC // GS
NSEL = 36
CW = NSEL * GS
BIG_I = 1 << 30

SC_CORES = 2
SC_SUBCORES = 16
NW = SC_CORES * SC_SUBCORES
ROWS_W = (B * G) // NW
PW = (B * G * K) // NW
GW = G // (NW // B)


def _k1_body(xt_ref, c_ref, d2_ref, gsel_ref, srow_ref, q_ref):
    b = pl.program_id(0)
    gb = pl.program_id(1)
    c = c_ref[0]
    c0 = c[:, 0:1]
    c1 = c[:, 1:2]
    c2 = c[:, 2:3]
    csq = c0 * c0 + c1 * c1 + c2 * c2
    cb16 = c.astype(jnp.bfloat16)
    inf = jnp.float32(jnp.inf)

    rowi = jax.lax.broadcasted_iota(jnp.int32, (NC, GPC), 0)
    coli = jax.lax.broadcasted_iota(jnp.int32, (NC, GPC), 1)
    sel = (rowi == coli * GS).astype(jnp.float32)

    mins = []
    for j in range(NCH):
        xc = xt_ref[0, j]
        x0 = xc[0:1, :]
        x1 = xc[1:2, :]
        x2 = xc[2:3, :]
        xsq = x0 * x0 + x1 * x1 + x2 * x2
        dot = jax.lax.dot_general(
            cb16, xc.astype(jnp.bfloat16), (((1,), (0,)), ((), ())),
            preferred_element_type=jnp.float32)
        d2 = (csq - 2.0 * dot) + xsq
        for h in range(NC // 128):
            d2_ref[0, j * (NC // 128) + h, :, :] = d2[:, h * 128:(h + 1) * 128]
        m = d2
        for sh in (1, 2, 4, 8, 16):
            m = jnp.minimum(m, pltpu.roll(m, NC - sh, 1))
        mins.append(jax.lax.dot_general(
            m, sel, (((1,), (0,)), ((), ())),
            preferred_element_type=jnp.float32,
            precision=jax.lax.Precision.HIGHEST))
    gmin = jnp.concatenate(mins, axis=1)

    cid = jax.lax.broadcasted_iota(jnp.int32, (BG, NG), 1)
    kiota = jax.lax.broadcasted_iota(jnp.int32, (BG, NSEL), 1)

    def select_grp(k, carry):
        mprev, iprev, acc = carry
        valid = (gmin > mprev) | ((gmin == mprev) & (cid > iprev))
        dm = jnp.where(valid, gmin, inf)
        mc = jnp.min(dm, axis=1, keepdims=True)
        ic = jnp.min(jnp.where(dm == mc, cid, BIG_I), axis=1, keepdims=True)
        acc = jnp.where(kiota == k, jnp.broadcast_to(ic, (BG, NSEL)), acc)
        return (mc, ic, acc)

    mprev0 = jnp.full((BG, 1), -jnp.inf, jnp.float32)
    iprev0 = jnp.full((BG, 1), -1, jnp.int32)
    acc0 = jnp.zeros((BG, NSEL), jnp.int32)
    _, _, acc = jax.lax.fori_loop(0, NSEL, select_grp, (mprev0, iprev0, acc0))
    gsel_ref[0] = acc
    g_row = gb * BG + jax.lax.broadcasted_iota(jnp.int32, (BG, NSEL), 0)
    srow_ref[0] = (b * (N // 128) + (acc >> 2)) * G + g_row
    q_ref[0] = acc & 3


def _k1_call(xt_c, center):
    return pl.pallas_call(
        _k1_body,
        grid=(B, G // BG),
        in_specs=[
            pl.BlockSpec((1, NCH, 3, NC), lambda b, g: (b, 0, 0, 0)),
            pl.BlockSpec((1, BG, 3), lambda b, g: (b, g, 0)),
        ],
        out_specs=[
            pl.BlockSpec((1, N // 128, BG, 128), lambda b, g: (b, 0, g, 0)),
            pl.BlockSpec((1, BG, NSEL), lambda b, g: (b, g, 0)),
            pl.BlockSpec((1, BG, NSEL), lambda b, g: (b, g, 0)),
            pl.BlockSpec((1, BG, NSEL), lambda b, g: (b, g, 0)),
        ],
        out_shape=[
            jax.ShapeDtypeStruct((B, N // 128, G, 128), jnp.float32),
            jax.ShapeDtypeStruct((B, G, NSEL), jnp.int32),
            jax.ShapeDtypeStruct((B, G, NSEL), jnp.int32),
            jax.ShapeDtypeStruct((B, G, NSEL), jnp.int32),
        ],
        compiler_params=pltpu.CompilerParams(
            dimension_semantics=("parallel", "parallel"),
        ),
    )(xt_c, center)


def _k2_body(tab_hbm, idx_hbm, q_hbm, out_hbm, idx_v, q_v,
             dest_0, dest_1, dest_2, dest_3,
             stg_0, stg_1, stg_2, stg_3,
             sem_0, sem_1, sem_2, sem_3,
             osem_0, osem_1, osem_2, osem_3):
    cid = jax.lax.axis_index("c")
    sid = jax.lax.axis_index("s")
    wid = sid * SC_CORES + cid
    rounds = (ROWS_W * NSEL) // 128
    nw = ROWS_W * NSEL
    dests = (dest_0, dest_1, dest_2, dest_3)
    sems = (sem_0, sem_1, sem_2, sem_3)
    stgs = (stg_0, stg_1, stg_2, stg_3)
    osems = (osem_0, osem_1, osem_2, osem_3)

    pltpu.sync_copy(idx_hbm.at[pl.ds(wid * nw, nw)], idx_v)
    pltpu.sync_copy(q_hbm.at[pl.ds(wid * nw, nw)], q_v)

    lanes = jax.lax.iota(jnp.int32, 16)

    def start(r, i):
        off = jnp.minimum(r, rounds - 1) * 128
        return pltpu.async_copy(
            tab_hbm.at[idx_v.at[pl.ds(off, 128)]], dests[i], sems[i])

    def trim(r, dest, stg):
        for sg in range(8):
            slots = sg * 16 + lanes
            q16 = q_v[pl.ds(r * 128 + sg * 16, 16)]
            col0 = q16 * GS
            base = slots * GS
            for s in range(GS):
                vals = plsc.load_gather(dest, [slots, col0 + s])
                plsc.store_scatter(stg, [base + s], vals)

    for i in range(4):
        start(jnp.int32(i), i)

    def quad(r4, carry):
        r0 = r4 * 4
        for i in range(4):
            r = r0 + i
            pltpu.make_async_copy(
                tab_hbm.at[idx_v.at[pl.ds(jnp.minimum(r, rounds - 1) * 128,
                                          128)]],
                dests[i], sems[i]).wait()
            @pl.when(r4 > 0)
            def _():
                pltpu.make_async_copy(
                    stgs[i],
                    out_hbm.at[pl.ds((wid * rounds + r - 4) * 128 * GS,
                                     128 * GS)],
                    osems[i]).wait()
            trim(r, dests[i], stgs[i])
            pltpu.async_copy(
                stgs[i],
                out_hbm.at[pl.ds((wid * rounds + r) * 128 * GS, 128 * GS)],
                osems[i])
            start(r + 4, i)
        return carry

    jax.lax.fori_loop(0, rounds // 4, quad, 0)
    for i in range(4):
        pltpu.make_async_copy(
            tab_hbm.at[idx_v.at[pl.ds((rounds - 1) * 128, 128)]],
            dests[i], sems[i]).wait()
        pltpu.make_async_copy(
            stgs[i],
            out_hbm.at[pl.ds((wid * rounds + rounds - 4 + i) * 128 * GS,
                             128 * GS)],
            osems[i]).wait()


@functools.cache
def _k2_call():
    return pl.kernel(
        _k2_body,
        out_type=jax.ShapeDtypeStruct((B * G * CW,), jnp.float32),
        mesh=plsc.VectorSubcoreMesh(
            core_axis_name="c", subcore_axis_name="s",
            num_cores=SC_CORES, num_subcores=SC_SUBCORES),
        compiler_params=pltpu.CompilerParams(needs_layout_passes=False),
        scratch_types=[
            pltpu.VMEM((ROWS_W * NSEL,), jnp.int32),
            pltpu.VMEM((ROWS_W * NSEL,), jnp.int32),
            pltpu.VMEM((128, 128), jnp.float32),
            pltpu.VMEM((128, 128), jnp.float32),
            pltpu.VMEM((128, 128), jnp.float32),
            pltpu.VMEM((128, 128), jnp.float32),
            pltpu.VMEM((128 * GS,), jnp.float32),
            pltpu.VMEM((128 * GS,), jnp.float32),
            pltpu.VMEM((128 * GS,), jnp.float32),
            pltpu.VMEM((128 * GS,), jnp.float32),
            pltpu.SemaphoreType.DMA,
            pltpu.SemaphoreType.DMA,
            pltpu.SemaphoreType.DMA,
            pltpu.SemaphoreType.DMA,
            pltpu.SemaphoreType.DMA,
            pltpu.SemaphoreType.DMA,
            pltpu.SemaphoreType.DMA,
            pltpu.SemaphoreType.DMA,
        ],
    )


def _k3_body(cand_ref, gsel_ref, idx_ref, orig_ref):
    inf = jnp.float32(jnp.inf)
    gsel = gsel_ref[0]
    siota = jax.lax.broadcasted_iota(jnp.int32, (BG3, GS), 1)
    for t in range(NSEL):
        orig_ref[:, t * GS:(t + 1) * GS] = gsel[:, t:t + 1] * GS + siota

    cand = cand_ref[0]
    orig = orig_ref[...]
    kiota = jax.lax.broadcasted_iota(jnp.int32, (BG3, K), 1)

    def select_k(k, carry):
        mprev, iprev, acc = carry
        valid = (cand > mprev) | ((cand == mprev) & (orig > iprev))
        dm = jnp.where(valid, cand, inf)
        mc = jnp.min(dm, axis=1, keepdims=True)
        ic = jnp.min(jnp.where(dm == mc, orig, BIG_I), axis=1, keepdims=True)
        acc = jnp.where(kiota == k, jnp.broadcast_to(ic, (BG3, K)), acc)
        return (mc, ic, acc)

    mprev0 = jnp.full((BG3, 1), -jnp.inf, jnp.float32)
    iprev0 = jnp.full((BG3, 1), -1, jnp.int32)
    acc0 = jnp.zeros((BG3, K), jnp.int32)
    _, _, acc = jax.lax.fori_loop(0, K, select_k, (mprev0, iprev0, acc0))
    idx_ref[0] = acc


def _k3_call(cand, gsel):
    return pl.pallas_call(
        _k3_body,
        grid=(B, G // BG3),
        in_specs=[
            pl.BlockSpec((1, BG3, CW), lambda b, g: (b, g, 0)),
            pl.BlockSpec((1, BG3, NSEL), lambda b, g: (b, g, 0)),
        ],
        out_specs=pl.BlockSpec((1, BG3, K), lambda b, g: (b, g, 0)),
        out_shape=jax.ShapeDtypeStruct((B, G, K), jnp.int32),
        scratch_shapes=[pltpu.VMEM((BG3, CW), jnp.int32)],
        compiler_params=pltpu.CompilerParams(
            dimension_semantics=("parallel", "parallel"),
        ),
    )(cand, gsel)


def _k4_body(xyz_hbm, idx_hbm, cen_hbm, out_hbm, xyz_v, idx_v, cen_v, out_v):
    cid = jax.lax.axis_index("c")
    sid = jax.lax.axis_index("s")
    wid = sid * SC_CORES + cid
    b = wid // (NW // B)
    gc = wid % (NW // B)

    pltpu.sync_copy(xyz_hbm.at[pl.ds(b * (N * 3), N * 3)], xyz_v)
    pltpu.sync_copy(idx_hbm.at[pl.ds(wid * PW, PW)], idx_v)
    pltpu.sync_copy(
        cen_hbm.at[pl.ds(b * (G * 3) + gc * (GW * 3), GW * 3)], cen_v)

    lanes = jax.lax.iota(jnp.int32, 16)

    def step(i, carry):
        base = i * 16
        iv = idx_v[pl.ds(base, 16)]
        lf = base + lanes
        g3 = jax.lax.shift_right_logical(lf, 5) * 3
        a3 = iv * 3
        o3 = lf * 3
        for d in range(3):
            p = plsc.load_gather(xyz_v, [a3 + d])
            c = plsc.load_gather(cen_v, [g3 + d])
            plsc.store_scatter(out_v, [o3 + d], p - c)
        return carry

    jax.lax.fori_loop(0, PW // 16, step, 0)

    pltpu.sync_copy(out_v, out_hbm.at[pl.ds(wid * (PW * 3), PW * 3)])


@functools.cache
def _k4_call():
    return pl.kernel(
        _k4_body,
        out_type=jax.ShapeDtypeStruct((B * G * K * 3,), jnp.float32),
        mesh=plsc.VectorSubcoreMesh(
            core_axis_name="c", subcore_axis_name="s",
            num_cores=SC_CORES, num_subcores=SC_SUBCORES),
        compiler_params=pltpu.CompilerParams(needs_layout_passes=False),
        scratch_types=[
            pltpu.VMEM((N * 3,), jnp.float32),
            pltpu.VMEM((PW,), jnp.int32),
            pltpu.VMEM((GW * 3,), jnp.float32),
            pltpu.VMEM((PW * 3,), jnp.float32),
        ],
    )


@jax.jit
def kernel(xyz, center):
    xt = jnp.swapaxes(xyz, 1, 2)
    xt_c = jnp.swapaxes(xt.reshape(B, 3, NCH, NC), 1, 2)
    d2lin, gsel, srow, q = _k1_call(xt_c, center)
    cand = _k2_call()(d2lin.reshape(B * (N // 128) * G, 128),
                      srow.reshape(B * G * NSEL),
                      q.reshape(B * G * NSEL))
    cand = cand.reshape(B, G, CW)
    idx = _k3_call(cand, gsel)
    out = _k4_call()(xyz.reshape(B * N * 3),
                     idx.reshape(B * G * K),
                     center.reshape(B * G * 3))
    return out.reshape(B, G, K, 3)
